# baseline (speedup 1.0000x reference)
import functools

import jax
import jax.numpy as jnp
from jax import lax
from jax.experimental import pallas as pl
from jax.experimental.pallas import tpu as pltpu
from jax.experimental.pallas import tpu_sc as plsc

D = 64
DP = 128
CHUNK = 512
NSUB = 2
SUB = CHUNK // NSUB


@functools.cache
def _sc_info():
    info = plsc.get_sparse_core_info()
    return info.num_cores, info.num_subcores


@functools.cache
def _make_reformat(V):
    NC, NS = _sc_info()
    NW = NC * NS
    n_cols = V // DP
    tail = V - n_cols * DP
    per_w = (n_cols + NW - 1) // NW
    mesh = plsc.VectorSubcoreMesh(core_axis_name="c", subcore_axis_name="s")

    @functools.partial(
        pl.kernel,
        mesh=mesh,
        out_type=jax.ShapeDtypeStruct((V // 2, DP), jnp.float32),
        compiler_params=pltpu.CompilerParams(
            use_tc_tiling_on_sc=True, needs_layout_passes=False
        ),
        scratch_types=[
            pltpu.VMEM((D, DP), jnp.float32),
            pltpu.VMEM((D, DP), jnp.float32),
        ],
    )
    def reformat_kernel(wt_hbm, tail_hbm, out_hbm, ib, cb):
        wid = lax.axis_index("s") * NC + lax.axis_index("c")
        lanes = lax.broadcasted_iota(jnp.int32, (16,), 0)

        if tail:
            @pl.when(wid == 0)
            def _():
                pltpu.sync_copy(tail_hbm, cb.at[pl.ds(0, tail // 2), :])
                pltpu.sync_copy(
                    cb.at[pl.ds(0, tail // 2), :],
                    out_hbm.at[pl.ds(n_cols * (DP // 2), tail // 2)],
                )

        def transpose_block(width):
            def jbody(j, _):
                for half in range(2):
                    col = 2 * j + half
                    cvec = jnp.full((16,), col, jnp.int32)
                    for q in range(4):
                        vec = plsc.load_gather(ib, [lanes + 16 * q, cvec])
                        cb[j, pl.ds(64 * half + 16 * q, 16)] = vec
                return 0

            lax.fori_loop(0, width // 2, jbody, 0)

        def body(t, _):
            tc = wid * per_w + t

            @pl.when(tc < n_cols)
            def _():
                c0 = tc * DP
                pltpu.sync_copy(wt_hbm.at[:, pl.ds(c0, DP)], ib)
                transpose_block(DP)
                pltpu.sync_copy(cb, out_hbm.at[pl.ds(tc * (DP // 2), DP // 2)])

            return 0

        lax.fori_loop(0, per_w, body, 0)

    return reformat_kernel


@functools.cache
def _make_gather(B):
    NC, NS = _sc_info()
    NW = NC * NS
    assert B % (8 * NW) == 0
    b_per_w = B // NW
    assert b_per_w % CHUNK == 0
    n_chunks = b_per_w // CHUNK
    n_pairs = n_chunks // 2
    assert n_chunks % 2 == 0
    mesh = plsc.VectorSubcoreMesh(core_axis_name="c", subcore_axis_name="s")

    @functools.partial(
        pl.kernel,
        mesh=mesh,
        out_type=jax.ShapeDtypeStruct((B, DP), jnp.float32),
        compiler_params=pltpu.CompilerParams(use_tc_tiling_on_sc=False),
        scratch_types=[
            pltpu.VMEM((b_per_w,), jnp.int32),
            pltpu.VMEM((CHUNK, D), jnp.float32),
            pltpu.VMEM((CHUNK, D), jnp.float32),
            pltpu.SemaphoreType.DMA,
            pltpu.SemaphoreType.DMA,
            pltpu.SemaphoreType.DMA,
            pltpu.SemaphoreType.DMA,
        ],
    )
    def gather_kernel(table_hbm, idx_hbm, out_hbm, idx_v, buf0, buf1, g0, g1, w0, w1):
        wid = lax.axis_index("s") * NC + lax.axis_index("c")
        base = wid * b_per_w
        pltpu.sync_copy(idx_hbm.at[pl.ds(base, b_per_w)], idx_v)

        def fire_gather(j, buf, sem):
            for k in range(NSUB):
                pltpu.async_copy(
                    table_hbm.at[idx_v.at[pl.ds(j * CHUNK + k * SUB, SUB)]],
                    buf.at[pl.ds(k * SUB, SUB)],
                    sem,
                )

        def wait_gather(buf, sem):
            pltpu.make_async_copy(
                table_hbm.at[idx_v.at[pl.ds(0, CHUNK)]], buf, sem
            ).wait()

        def fire_write(j, buf, sem):
            pltpu.async_copy(
                buf,
                out_hbm.at[pl.ds(base + j * CHUNK, CHUNK), pl.ds(0, D)],
                sem,
            )

        def wait_write(buf, sem):
            pltpu.make_async_copy(
                buf,
                out_hbm.at[pl.ds(base, CHUNK), pl.ds(0, D)],
                sem,
            ).wait()

        fire_gather(0, buf0, g0)

        def body(p, _):
            j0 = 2 * p
            j1 = j0 + 1

            @pl.when(p > 0)
            def _():
                wait_write(buf1, w1)

            fire_gather(j1, buf1, g1)
            wait_gather(buf0, g0)
            fire_write(j0, buf0, w0)
            wait_gather(buf1, g1)
            fire_write(j1, buf1, w1)

            @pl.when(p < n_pairs - 1)
            def _():
                wait_write(buf0, w0)
                fire_gather(j0 + 2, buf0, g0)

            return 0

        lax.fori_loop(0, n_pairs, body, 0)
        wait_write(buf0, w0)
        wait_write(buf1, w1)

    return gather_kernel


def kernel(x, embedding_weight):
    B = x.size
    V, d = embedding_weight.shape
    flat_idx = x.reshape(B).astype(jnp.int32)
    n_cols = V // DP
    tail_packed = embedding_weight[n_cols * DP :].reshape(-1, DP)
    packed = _make_reformat(V)(embedding_weight.T, tail_packed)
    table = packed.reshape(V, d)
    out = _make_gather(B)(table, flat_idx)
    return out[:, :d].reshape(x.shape + (d,))


# speedup vs baseline: 2.5088x; 2.5088x over previous
import functools

import jax
import jax.numpy as jnp
from jax import lax
from jax.experimental import pallas as pl
from jax.experimental.pallas import tpu as pltpu
from jax.experimental.pallas import tpu_sc as plsc

D = 64
DP = 128
CHUNK = 512
NSUB = 2
SUB = CHUNK // NSUB


@functools.cache
def _make_gather(B):
    info = plsc.get_sparse_core_info()
    NC, NS = info.num_cores, info.num_subcores
    NW = NC * NS
    assert B % (8 * NW) == 0
    b_per_w = B // NW
    assert b_per_w % CHUNK == 0
    n_chunks = b_per_w // CHUNK
    n_pairs = n_chunks // 2
    assert n_chunks % 2 == 0
    mesh = plsc.VectorSubcoreMesh(core_axis_name="c", subcore_axis_name="s")

    @functools.partial(
        pl.kernel,
        mesh=mesh,
        out_type=jax.ShapeDtypeStruct((B, DP), jnp.float32),
        compiler_params=pltpu.CompilerParams(use_tc_tiling_on_sc=False),
        scratch_types=[
            pltpu.VMEM((b_per_w,), jnp.int32),
            pltpu.VMEM((CHUNK, D), jnp.float32),
            pltpu.VMEM((CHUNK, D), jnp.float32),
            pltpu.SemaphoreType.DMA,
            pltpu.SemaphoreType.DMA,
            pltpu.SemaphoreType.DMA,
            pltpu.SemaphoreType.DMA,
        ],
    )
    def gather_kernel(table_hbm, idx_hbm, out_hbm, idx_v, buf0, buf1, g0, g1, w0, w1):
        wid = lax.axis_index("s") * NC + lax.axis_index("c")
        base = wid * b_per_w
        pltpu.sync_copy(idx_hbm.at[pl.ds(base, b_per_w)], idx_v)

        def fire_gather(j, buf, sem):
            for k in range(NSUB):
                pltpu.async_copy(
                    table_hbm.at[idx_v.at[pl.ds(j * CHUNK + k * SUB, SUB)]],
                    buf.at[pl.ds(k * SUB, SUB)],
                    sem,
                )

        def wait_gather(buf, sem):
            pltpu.make_async_copy(
                table_hbm.at[idx_v.at[pl.ds(0, CHUNK)]], buf, sem
            ).wait()

        def fire_write(j, buf, sem):
            pltpu.async_copy(
                buf,
                out_hbm.at[pl.ds(base + j * CHUNK, CHUNK), pl.ds(0, D)],
                sem,
            )

        def wait_write(buf, sem):
            pltpu.make_async_copy(
                buf,
                out_hbm.at[pl.ds(base, CHUNK), pl.ds(0, D)],
                sem,
            ).wait()

        fire_gather(0, buf0, g0)

        def body(p, _):
            j0 = 2 * p
            j1 = j0 + 1

            @pl.when(p > 0)
            def _():
                wait_write(buf1, w1)

            fire_gather(j1, buf1, g1)
            wait_gather(buf0, g0)
            fire_write(j0, buf0, w0)
            wait_gather(buf1, g1)
            fire_write(j1, buf1, w1)

            @pl.when(p < n_pairs - 1)
            def _():
                wait_write(buf0, w0)
                fire_gather(j0 + 2, buf0, g0)

            return 0

        lax.fori_loop(0, n_pairs, body, 0)
        wait_write(buf0, w0)
        wait_write(buf1, w1)

    return gather_kernel


def kernel(x, embedding_weight):
    B = x.size
    V, d = embedding_weight.shape
    flat_idx = x.reshape(B).astype(jnp.int32) * 2
    wp = jnp.pad(embedding_weight, ((0, 0), (0, DP - d)))
    table2 = wp.reshape(2 * V, d)
    out = _make_gather(B)(table2, flat_idx)
    return out[:, :d].reshape(x.shape + (d,))
